# Initial kernel scaffold; baseline (speedup 1.0000x reference)
import jax
import jax.numpy as jnp
from jax.experimental import pallas as pl


def kernel(x, edge_index, We1, be1, We2, be2, Wg1, bg1, Wg2, bg2, Wp1, bp1, Wp2, bp2):
    raise NotImplementedError("write your pallas kernel here")


# speedup vs baseline: 2.4308x; 2.4308x over previous
import functools

import jax
import jax.numpy as jnp
from jax import lax
from jax.experimental import pallas as pl
from jax.experimental.pallas import tpu as pltpu
from jax.experimental.pallas import tpu_sc as plsc

N = 100000
NH = 50000
E = 1600000
CH = 128
TPC = 784
EP = 16 * TPC * CH
PAD = EP - E
NROW = EP // CH
STG = 16
NB = TPC // STG
R = 52096
TRASH = 50000
RPS = R // 16
ZB = 296

BLK = 2000
GRIDH = NH // BLK


def _mesh():
    return plsc.VectorSubcoreMesh(
        core_axis_name="c", subcore_axis_name="s", num_cores=2, num_subcores=16
    )


def _zero_init(zb_v, acc_sh, s):
    zero = jnp.zeros((16,), jnp.float32)

    def fill_zero(i, carry):
        zb_v[i, :] = zero
        return carry

    lax.fori_loop(0, ZB, fill_zero, 0)

    def zcopy(k, carry):
        pltpu.sync_copy(
            zb_v, acc_sh.at[pl.ds(pl.multiple_of(s * RPS + k * ZB, 8), ZB), :]
        )
        return carry

    lax.fori_loop(0, 11, zcopy, 0)


def _sc_degree(dm):
    @functools.partial(
        pl.kernel,
        out_type=jax.ShapeDtypeStruct((2 * R, 16), jnp.float32),
        mesh=_mesh(),
        compiler_params=pltpu.CompilerParams(use_tc_tiling_on_sc=False),
        scratch_types=[
            pltpu.VMEM((STG, CH), jnp.int32),
            pltpu.VMEM((CH, 16), jnp.float32),
            pltpu.VMEM((ZB, 16), jnp.float32),
            pltpu.VMEM_SHARED((R, 16), jnp.float32),
        ],
    )
    def body(dm_hbm, out_hbm, idx_v, ones_v, zb_v, acc_sh):
        c = lax.axis_index("c")
        s = lax.axis_index("s")
        one = jnp.ones((16,), jnp.float32)

        def fill_ones(i, carry):
            ones_v[i, :] = one
            return carry

        lax.fori_loop(0, CH, fill_ones, 0)
        _zero_init(zb_v, acc_sh, s)
        plsc.subcore_barrier()

        def stage(b, carry):
            row0 = pl.multiple_of(c * NROW + s * TPC + b * STG, 8)
            pltpu.sync_copy(dm_hbm.at[pl.ds(row0, STG), :], idx_v)
            for j in range(STG):
                pltpu.sync_copy(ones_v, acc_sh.at[idx_v.at[j]], add=True)
            return carry

        lax.fori_loop(0, NB, stage, 0)
        plsc.subcore_barrier()
        pltpu.sync_copy(
            acc_sh.at[pl.ds(pl.multiple_of(s * RPS, 8), RPS), :],
            out_hbm.at[pl.ds(pl.multiple_of(c * R + s * RPS, 8), RPS), :],
        )

    return body(dm)


def _sc_scatter(sm, dm, g2n, h):
    @functools.partial(
        pl.kernel,
        out_type=jax.ShapeDtypeStruct((2 * R, 16), jnp.float32),
        mesh=_mesh(),
        compiler_params=pltpu.CompilerParams(use_tc_tiling_on_sc=False),
        scratch_types=[
            pltpu.VMEM((STG, CH), jnp.int32),
            pltpu.VMEM((STG, CH), jnp.int32),
            pltpu.VMEM((CH, 16), jnp.float32),
            pltpu.VMEM((ZB, 16), jnp.float32),
            pltpu.VMEM_SHARED((R, 16), jnp.float32),
            pltpu.SemaphoreType.DMA,
        ],
    )
    def body(sm_hbm, dm_hbm, g_hbm, out_hbm, sidx, didx, rows_v, zb_v, acc_sh, sem):
        c = lax.axis_index("c")
        s = lax.axis_index("s")
        _zero_init(zb_v, acc_sh, s)
        plsc.subcore_barrier()

        def stage(b, carry):
            srow = pl.multiple_of((2 * h + c) * NROW + s * TPC + b * STG, 8)
            drow = pl.multiple_of(h * NROW + s * TPC + b * STG, 8)
            pltpu.sync_copy(sm_hbm.at[pl.ds(srow, STG), :], sidx)
            pltpu.sync_copy(dm_hbm.at[pl.ds(drow, STG), :], didx)
            for j in range(STG):
                pltpu.async_copy(g_hbm.at[sidx.at[j]], rows_v, sem).wait()
                pltpu.sync_copy(rows_v, acc_sh.at[didx.at[j]], add=True)
            return carry

        lax.fori_loop(0, NB, stage, 0)
        plsc.subcore_barrier()
        pltpu.sync_copy(
            acc_sh.at[pl.ds(pl.multiple_of(s * RPS, 8), RPS), :],
            out_hbm.at[pl.ds(pl.multiple_of(c * R + s * RPS, 8), RPS), :],
        )

    return body(sm, dm, g2n)


def _tc_edges(sp2d, dp2d):
    EB = 112

    def body(s_ref, d_ref, sm_ref, dm_ref):
        s = s_ref[...]
        d = d_ref[...]
        in0 = d < NH
        in1 = jnp.logical_and(d >= NH, d < N)
        dm_ref[0] = jnp.where(in0, d, TRASH)
        dm_ref[1] = jnp.where(in1, d - NH, TRASH)
        for c in range(2):
            off = jnp.int32(c * N)
            sm_ref[0, c] = jnp.where(in0, s + off, off)
            sm_ref[1, c] = jnp.where(in1, s + off, off)

    return pl.pallas_call(
        body,
        grid=(NROW // EB,),
        in_specs=[
            pl.BlockSpec((EB, CH), lambda i: (i, 0)),
            pl.BlockSpec((EB, CH), lambda i: (i, 0)),
        ],
        out_specs=[
            pl.BlockSpec((2, 2, EB, CH), lambda i: (0, 0, i, 0)),
            pl.BlockSpec((2, EB, CH), lambda i: (0, i, 0)),
        ],
        out_shape=[
            jax.ShapeDtypeStruct((2, 2, NROW, CH), jnp.int32),
            jax.ShapeDtypeStruct((2, NROW, CH), jnp.int32),
        ],
    )(sp2d, dp2d)


def _tc_embed(x, We1, be1, We2, be2):
    def body(x_ref, w1_ref, b1_ref, w2_ref, b2_ref, o_ref):
        h = jnp.tanh(
            jnp.dot(x_ref[...], w1_ref[...], preferred_element_type=jnp.float32)
            + b1_ref[...]
        )
        o_ref[...] = jnp.tanh(
            jnp.dot(h, w2_ref[...], preferred_element_type=jnp.float32) + b2_ref[...]
        )

    return pl.pallas_call(
        body,
        grid=(N // BLK,),
        in_specs=[
            pl.BlockSpec((BLK, 6), lambda i: (i, 0)),
            pl.BlockSpec((6, 64), lambda i: (0, 0)),
            pl.BlockSpec((1, 64), lambda i: (0, 0)),
            pl.BlockSpec((64, 32), lambda i: (0, 0)),
            pl.BlockSpec((1, 32), lambda i: (0, 0)),
        ],
        out_specs=pl.BlockSpec((BLK, 32), lambda i: (i, 0)),
        out_shape=jax.ShapeDtypeStruct((N, 32), jnp.float32),
    )(x, We1, be1, We2, be2)


def _tc_prep(h0, degp, Wg1h):
    def body(h_ref, p_ref, w_ref, g_ref, dinv_ref):
        deg = p_ref[...] + 1.0
        dinv = lax.rsqrt(deg)
        t = jnp.dot(h_ref[...], w_ref[0], preferred_element_type=jnp.float32)
        g_ref[...] = (t * dinv)[None]
        dinv_ref[...] = dinv

    return pl.pallas_call(
        body,
        grid=(N // BLK, 2),
        in_specs=[
            pl.BlockSpec((BLK, 32), lambda i, c: (i, 0)),
            pl.BlockSpec((BLK, 1), lambda i, c: (i, 0)),
            pl.BlockSpec((1, 32, 16), lambda i, c: (c, 0, 0)),
        ],
        out_specs=[
            pl.BlockSpec((1, BLK, 16), lambda i, c: (c, i, 0)),
            pl.BlockSpec((BLK, 1), lambda i, c: (i, 0)),
        ],
        out_shape=[
            jax.ShapeDtypeStruct((2, N, 16), jnp.float32),
            jax.ShapeDtypeStruct((N, 1), jnp.float32),
        ],
    )(h0, degp, Wg1h)


def _tc_comb(acc, g, dinv, bg, Wh, h):
    def body(a_ref, g_ref, dinv_ref, bg_ref, w_ref, g2_ref):
        accv = jnp.concatenate([a_ref[0], a_ref[1]], axis=1)
        gv = jnp.concatenate([g_ref[0], g_ref[1]], axis=1)
        hv = jnp.maximum(dinv_ref[...] * (accv + gv) + bg_ref[...], 0.0)
        t = jnp.dot(hv, w_ref[0], preferred_element_type=jnp.float32)
        g2_ref[...] = (t * dinv_ref[...])[None]

    return pl.pallas_call(
        body,
        grid=(GRIDH, 2),
        in_specs=[
            pl.BlockSpec((2, BLK, 16), lambda i, c: (0, i, 0)),
            pl.BlockSpec((2, BLK, 16), lambda i, c: (0, h * GRIDH + i, 0)),
            pl.BlockSpec((BLK, 1), lambda i, c: (h * GRIDH + i, 0)),
            pl.BlockSpec((1, 32), lambda i, c: (0, 0)),
            pl.BlockSpec((1, 32, 16), lambda i, c: (c, 0, 0)),
        ],
        out_specs=pl.BlockSpec((1, BLK, 16), lambda i, c: (c, i, 0)),
        out_shape=jax.ShapeDtypeStruct((2, NH, 16), jnp.float32),
    )(acc, g, dinv, bg, Wh)


def _tc_final(acc, g, dinv, bg2, Wp1, bp1, Wp2, bp2, h):
    def body(a_ref, g_ref, dinv_ref, bg_ref, w1_ref, b1_ref, w2_ref, b2_ref, o_ref):
        accv = jnp.concatenate([a_ref[0], a_ref[1]], axis=1)
        gv = jnp.concatenate([g_ref[0], g_ref[1]], axis=1)
        h2 = jnp.maximum(dinv_ref[...] * (accv + gv) + bg_ref[...], 0.0)
        a1 = jnp.tanh(
            jnp.dot(h2, w1_ref[...], preferred_element_type=jnp.float32) + b1_ref[...]
        )
        o_ref[...] = jnp.tanh(
            jnp.dot(a1, w2_ref[...], preferred_element_type=jnp.float32) + b2_ref[...]
        )

    return pl.pallas_call(
        body,
        grid=(GRIDH,),
        in_specs=[
            pl.BlockSpec((2, BLK, 16), lambda i: (0, i, 0)),
            pl.BlockSpec((2, BLK, 16), lambda i: (0, h * GRIDH + i, 0)),
            pl.BlockSpec((BLK, 1), lambda i: (h * GRIDH + i, 0)),
            pl.BlockSpec((1, 32), lambda i: (0, 0)),
            pl.BlockSpec((32, 32), lambda i: (0, 0)),
            pl.BlockSpec((1, 32), lambda i: (0, 0)),
            pl.BlockSpec((32, 1), lambda i: (0, 0)),
            pl.BlockSpec((1, 1), lambda i: (0, 0)),
        ],
        out_specs=pl.BlockSpec((BLK, 1), lambda i: (i, 0)),
        out_shape=jax.ShapeDtypeStruct((NH, 1), jnp.float32),
    )(acc, g, dinv, bg2, Wp1, bp1, Wp2, bp2)


def kernel(x, edge_index, We1, be1, We2, be2, Wg1, bg1, Wg2, bg2, Wp1, bp1, Wp2, bp2):
    src = edge_index[0]
    dst = edge_index[1]
    sp2d = jnp.concatenate([src, jnp.zeros((PAD,), jnp.int32)]).reshape(NROW, CH)
    dp2d = jnp.concatenate([dst, jnp.full((PAD,), N, jnp.int32)]).reshape(NROW, CH)
    sm, dm = _tc_edges(sp2d, dp2d)
    sm = sm.reshape(4 * NROW, CH)
    dm = dm.reshape(2 * NROW, CH)

    Wg1h = Wg1.reshape(32, 2, 16).transpose(1, 0, 2)
    Wg2h = Wg2.reshape(32, 2, 16).transpose(1, 0, 2)

    degp = _sc_degree(dm).reshape(2, R, 16)
    deg = jnp.concatenate([degp[0, :NH, 0:1], degp[1, :NH, 0:1]], axis=0)
    h0 = _tc_embed(x, We1, be1.reshape(1, 64), We2, be2.reshape(1, 32))
    g1, dinv = _tc_prep(h0, deg, Wg1h)

    g1f = g1.reshape(2 * N, 16)
    acc1 = [_sc_scatter(sm, dm, g1f, h).reshape(2, R, 16) for h in range(2)]
    g2 = jnp.concatenate(
        [_tc_comb(acc1[h], g1, dinv, bg1.reshape(1, 32), Wg2h, h) for h in range(2)],
        axis=1,
    )

    g2f = g2.reshape(2 * N, 16)
    acc2 = [_sc_scatter(sm, dm, g2f, h).reshape(2, R, 16) for h in range(2)]
    outs = [
        _tc_final(acc2[h], g2, dinv, bg2.reshape(1, 32), Wp1, bp1.reshape(1, 32),
                  Wp2, bp2.reshape(1, 1), h)
        for h in range(2)
    ]
    return jnp.concatenate(outs, axis=0)


# speedup vs baseline: 2.4313x; 1.0002x over previous
import functools

import jax
import jax.numpy as jnp
from jax import lax
from jax.experimental import pallas as pl
from jax.experimental.pallas import tpu as pltpu
from jax.experimental.pallas import tpu_sc as plsc

N = 100000
NH = 50000
E = 1600000
CH = 128
TPC = 784
EP = 16 * TPC * CH
PAD = EP - E
NROW = EP // CH
STG = 16
NB = TPC // STG
NBUF = 8
DEPTH = 4
R = 52096
TRASH = 50000
RPS = R // 16
ZB = 296

BLK = 2000
GRIDH = NH // BLK


def _mesh():
    return plsc.VectorSubcoreMesh(
        core_axis_name="c", subcore_axis_name="s", num_cores=2, num_subcores=16
    )


def _zero_init(zb_v, acc_sh, s):
    zero = jnp.zeros((16,), jnp.float32)

    def fill_zero(i, carry):
        zb_v[i, :] = zero
        return carry

    lax.fori_loop(0, ZB, fill_zero, 0)

    def zcopy(k, carry):
        pltpu.sync_copy(
            zb_v, acc_sh.at[pl.ds(pl.multiple_of(s * RPS + k * ZB, 8), ZB), :]
        )
        return carry

    lax.fori_loop(0, 11, zcopy, 0)


def _sc_degree(dm):
    @functools.partial(
        pl.kernel,
        out_type=jax.ShapeDtypeStruct((2 * R, 16), jnp.float32),
        mesh=_mesh(),
        compiler_params=pltpu.CompilerParams(use_tc_tiling_on_sc=False),
        scratch_types=[
            pltpu.VMEM((STG, CH), jnp.int32),
            pltpu.VMEM((CH, 16), jnp.float32),
            pltpu.VMEM((ZB, 16), jnp.float32),
            pltpu.VMEM_SHARED((R, 16), jnp.float32),
        ],
    )
    def body(dm_hbm, out_hbm, idx_v, ones_v, zb_v, acc_sh):
        c = lax.axis_index("c")
        s = lax.axis_index("s")
        one = jnp.ones((16,), jnp.float32)

        def fill_ones(i, carry):
            ones_v[i, :] = one
            return carry

        lax.fori_loop(0, CH, fill_ones, 0)
        _zero_init(zb_v, acc_sh, s)
        plsc.subcore_barrier()

        def stage(b, carry):
            row0 = pl.multiple_of(c * NROW + s * TPC + b * STG, 8)
            pltpu.sync_copy(dm_hbm.at[pl.ds(row0, STG), :], idx_v)
            for j in range(STG):
                pltpu.sync_copy(ones_v, acc_sh.at[idx_v.at[j]], add=True)
            return carry

        lax.fori_loop(0, NB, stage, 0)
        plsc.subcore_barrier()
        pltpu.sync_copy(
            acc_sh.at[pl.ds(pl.multiple_of(s * RPS, 8), RPS), :],
            out_hbm.at[pl.ds(pl.multiple_of(c * R + s * RPS, 8), RPS), :],
        )

    return body(dm)


def _sc_scatter(sm, dm, g2n, h):
    @functools.partial(
        pl.kernel,
        out_type=jax.ShapeDtypeStruct((2 * R, 16), jnp.float32),
        mesh=_mesh(),
        compiler_params=pltpu.CompilerParams(use_tc_tiling_on_sc=False),
        scratch_types=[
            pltpu.VMEM((STG, CH), jnp.int32),
            pltpu.VMEM((STG, CH), jnp.int32),
            pltpu.VMEM((NBUF, CH, 16), jnp.float32),
            pltpu.VMEM((ZB, 16), jnp.float32),
            pltpu.VMEM_SHARED((R, 16), jnp.float32),
            pltpu.SemaphoreType.DMA,
        ],
    )
    def body(sm_hbm, dm_hbm, g_hbm, out_hbm, sidx, didx, rows_v, zb_v, acc_sh,
             gsem):
        c = lax.axis_index("c")
        s = lax.axis_index("s")
        _zero_init(zb_v, acc_sh, s)
        plsc.subcore_barrier()

        def gfire(j):
            pltpu.async_copy(g_hbm.at[sidx.at[j]], rows_v.at[j % NBUF], gsem)

        def gwait(j):
            pltpu.make_async_copy(
                g_hbm.at[sidx.at[j]], rows_v.at[j % NBUF], gsem
            ).wait()

        def stage(b, carry):
            srow = pl.multiple_of((2 * h + c) * NROW + s * TPC + b * STG, 8)
            drow = pl.multiple_of(h * NROW + s * TPC + b * STG, 8)
            pltpu.sync_copy(sm_hbm.at[pl.ds(srow, STG), :], sidx)
            pltpu.sync_copy(dm_hbm.at[pl.ds(drow, STG), :], didx)
            for j in range(DEPTH):
                gfire(j)
            for j in range(STG):
                gwait(j)
                if j + DEPTH < STG:
                    gfire(j + DEPTH)
                pltpu.sync_copy(
                    rows_v.at[j % NBUF], acc_sh.at[didx.at[j]], add=True
                )
            return carry

        lax.fori_loop(0, NB, stage, 0)
        plsc.subcore_barrier()
        pltpu.sync_copy(
            acc_sh.at[pl.ds(pl.multiple_of(s * RPS, 8), RPS), :],
            out_hbm.at[pl.ds(pl.multiple_of(c * R + s * RPS, 8), RPS), :],
        )

    return body(sm, dm, g2n)


def _tc_edges(sp2d, dp2d):
    EB = 112
    EROW = EP // 128

    def body(s_ref, d_ref, sm_ref, dm_ref):
        s = s_ref[...]
        d = d_ref[...]
        in0 = d < NH
        in1 = jnp.logical_and(d >= NH, d < N)
        dm_ref[0] = jnp.where(in0, d, TRASH)
        dm_ref[1] = jnp.where(in1, d - NH, TRASH)
        for c in range(2):
            off = jnp.int32(c * N)
            sm_ref[0, c] = jnp.where(in0, s + off, off)
            sm_ref[1, c] = jnp.where(in1, s + off, off)

    return pl.pallas_call(
        body,
        grid=(EROW // EB,),
        in_specs=[
            pl.BlockSpec((EB, 128), lambda i: (i, 0)),
            pl.BlockSpec((EB, 128), lambda i: (i, 0)),
        ],
        out_specs=[
            pl.BlockSpec((2, 2, EB, 128), lambda i: (0, 0, i, 0)),
            pl.BlockSpec((2, EB, 128), lambda i: (0, i, 0)),
        ],
        out_shape=[
            jax.ShapeDtypeStruct((2, 2, EROW, 128), jnp.int32),
            jax.ShapeDtypeStruct((2, EROW, 128), jnp.int32),
        ],
    )(sp2d, dp2d)


def _tc_embed(x, We1, be1, We2, be2):
    def body(x_ref, w1_ref, b1_ref, w2_ref, b2_ref, o_ref):
        h = jnp.tanh(
            jnp.dot(x_ref[...], w1_ref[...], preferred_element_type=jnp.float32)
            + b1_ref[...]
        )
        o_ref[...] = jnp.tanh(
            jnp.dot(h, w2_ref[...], preferred_element_type=jnp.float32) + b2_ref[...]
        )

    return pl.pallas_call(
        body,
        grid=(N // BLK,),
        in_specs=[
            pl.BlockSpec((BLK, 6), lambda i: (i, 0)),
            pl.BlockSpec((6, 64), lambda i: (0, 0)),
            pl.BlockSpec((1, 64), lambda i: (0, 0)),
            pl.BlockSpec((64, 32), lambda i: (0, 0)),
            pl.BlockSpec((1, 32), lambda i: (0, 0)),
        ],
        out_specs=pl.BlockSpec((BLK, 32), lambda i: (i, 0)),
        out_shape=jax.ShapeDtypeStruct((N, 32), jnp.float32),
    )(x, We1, be1, We2, be2)


def _tc_prep(h0, degp, Wg1h):
    def body(h_ref, p_ref, w_ref, g_ref, dinv_ref):
        deg = p_ref[...] + 1.0
        dinv = lax.rsqrt(deg)
        t = jnp.dot(h_ref[...], w_ref[0], preferred_element_type=jnp.float32)
        g_ref[...] = (t * dinv)[None]
        dinv_ref[...] = dinv

    return pl.pallas_call(
        body,
        grid=(N // BLK, 2),
        in_specs=[
            pl.BlockSpec((BLK, 32), lambda i, c: (i, 0)),
            pl.BlockSpec((BLK, 1), lambda i, c: (i, 0)),
            pl.BlockSpec((1, 32, 16), lambda i, c: (c, 0, 0)),
        ],
        out_specs=[
            pl.BlockSpec((1, BLK, 16), lambda i, c: (c, i, 0)),
            pl.BlockSpec((BLK, 1), lambda i, c: (i, 0)),
        ],
        out_shape=[
            jax.ShapeDtypeStruct((2, N, 16), jnp.float32),
            jax.ShapeDtypeStruct((N, 1), jnp.float32),
        ],
    )(h0, degp, Wg1h)


def _tc_comb(acc, g, dinv, bg, Wh, h):
    def body(a_ref, g_ref, dinv_ref, bg_ref, w_ref, g2_ref):
        accv = jnp.concatenate([a_ref[0], a_ref[1]], axis=1)
        gv = jnp.concatenate([g_ref[0], g_ref[1]], axis=1)
        hv = jnp.maximum(dinv_ref[...] * (accv + gv) + bg_ref[...], 0.0)
        t = jnp.dot(hv, w_ref[0], preferred_element_type=jnp.float32)
        g2_ref[...] = (t * dinv_ref[...])[None]

    return pl.pallas_call(
        body,
        grid=(GRIDH, 2),
        in_specs=[
            pl.BlockSpec((2, BLK, 16), lambda i, c: (0, i, 0)),
            pl.BlockSpec((2, BLK, 16), lambda i, c: (0, h * GRIDH + i, 0)),
            pl.BlockSpec((BLK, 1), lambda i, c: (h * GRIDH + i, 0)),
            pl.BlockSpec((1, 32), lambda i, c: (0, 0)),
            pl.BlockSpec((1, 32, 16), lambda i, c: (c, 0, 0)),
        ],
        out_specs=pl.BlockSpec((1, BLK, 16), lambda i, c: (c, i, 0)),
        out_shape=jax.ShapeDtypeStruct((2, NH, 16), jnp.float32),
    )(acc, g, dinv, bg, Wh)


def _tc_final(acc, g, dinv, bg2, Wp1, bp1, Wp2, bp2, h):
    def body(a_ref, g_ref, dinv_ref, bg_ref, w1_ref, b1_ref, w2_ref, b2_ref, o_ref):
        accv = jnp.concatenate([a_ref[0], a_ref[1]], axis=1)
        gv = jnp.concatenate([g_ref[0], g_ref[1]], axis=1)
        h2 = jnp.maximum(dinv_ref[...] * (accv + gv) + bg_ref[...], 0.0)
        a1 = jnp.tanh(
            jnp.dot(h2, w1_ref[...], preferred_element_type=jnp.float32) + b1_ref[...]
        )
        o_ref[...] = jnp.tanh(
            jnp.dot(a1, w2_ref[...], preferred_element_type=jnp.float32) + b2_ref[...]
        )

    return pl.pallas_call(
        body,
        grid=(GRIDH,),
        in_specs=[
            pl.BlockSpec((2, BLK, 16), lambda i: (0, i, 0)),
            pl.BlockSpec((2, BLK, 16), lambda i: (0, h * GRIDH + i, 0)),
            pl.BlockSpec((BLK, 1), lambda i: (h * GRIDH + i, 0)),
            pl.BlockSpec((1, 32), lambda i: (0, 0)),
            pl.BlockSpec((32, 32), lambda i: (0, 0)),
            pl.BlockSpec((1, 32), lambda i: (0, 0)),
            pl.BlockSpec((32, 1), lambda i: (0, 0)),
            pl.BlockSpec((1, 1), lambda i: (0, 0)),
        ],
        out_specs=pl.BlockSpec((BLK, 1), lambda i: (i, 0)),
        out_shape=jax.ShapeDtypeStruct((NH, 1), jnp.float32),
    )(acc, g, dinv, bg2, Wp1, bp1, Wp2, bp2)


def kernel(x, edge_index, We1, be1, We2, be2, Wg1, bg1, Wg2, bg2, Wp1, bp1, Wp2, bp2):
    src = edge_index[0]
    dst = edge_index[1]
    sp2d = jnp.concatenate([src, jnp.zeros((PAD,), jnp.int32)]).reshape(EP // 128, 128)
    dp2d = jnp.concatenate([dst, jnp.full((PAD,), N, jnp.int32)]).reshape(EP // 128, 128)
    sm, dm = _tc_edges(sp2d, dp2d)
    sm = sm.reshape(4 * NROW, CH)
    dm = dm.reshape(2 * NROW, CH)

    Wg1h = Wg1.reshape(32, 2, 16).transpose(1, 0, 2)
    Wg2h = Wg2.reshape(32, 2, 16).transpose(1, 0, 2)

    degp = _sc_degree(dm).reshape(2, R, 16)
    deg = jnp.concatenate([degp[0, :NH, 0:1], degp[1, :NH, 0:1]], axis=0)
    h0 = _tc_embed(x, We1, be1.reshape(1, 64), We2, be2.reshape(1, 32))
    g1, dinv = _tc_prep(h0, deg, Wg1h)

    g1f = g1.reshape(2 * N, 16)
    acc1 = [_sc_scatter(sm, dm, g1f, h).reshape(2, R, 16) for h in range(2)]
    g2 = jnp.concatenate(
        [_tc_comb(acc1[h], g1, dinv, bg1.reshape(1, 32), Wg2h, h) for h in range(2)],
        axis=1,
    )

    g2f = g2.reshape(2 * N, 16)
    acc2 = [_sc_scatter(sm, dm, g2f, h).reshape(2, R, 16) for h in range(2)]
    outs = [
        _tc_final(acc2[h], g2, dinv, bg2.reshape(1, 32), Wp1, bp1.reshape(1, 32),
                  Wp2, bp2.reshape(1, 1), h)
        for h in range(2)
    ]
    return jnp.concatenate(outs, axis=0)


# speedup vs baseline: 24.3775x; 10.0263x over previous
import functools

import jax
import jax.numpy as jnp
from jax import lax
from jax.experimental import pallas as pl
from jax.experimental.pallas import tpu as pltpu
from jax.experimental.pallas import tpu_sc as plsc

N = 100000
E = 1600000
CH = 128
TPC = 784
EP = 16 * TPC * CH
PAD = EP - E
NROW = EP // CH
STG = 16
NB = TPC // STG
NBUF = 8
DEPTH = 4
DTPW = NROW // 32
DSTG = 8
DNB = DTPW // DSTG
RN = 100096
RPS = RN // 16
ZB = 368

BLK = 2000
GRID = N // BLK


def _mesh():
    return plsc.VectorSubcoreMesh(
        core_axis_name="c", subcore_axis_name="s", num_cores=2, num_subcores=16
    )


def _zero_init(zb_v, acc_sh, s):
    zero = jnp.zeros((16,), jnp.float32)

    def fill_zero(i, carry):
        zb_v[i, :] = zero
        return carry

    lax.fori_loop(0, ZB, fill_zero, 0)

    def zcopy(k, carry):
        pltpu.sync_copy(
            zb_v, acc_sh.at[pl.ds(pl.multiple_of(s * RPS + k * ZB, 8), ZB), :]
        )
        return carry

    lax.fori_loop(0, 17, zcopy, 0)


def _writeback(acc_sh, out_hbm, c, s):
    pltpu.sync_copy(
        acc_sh.at[pl.ds(pl.multiple_of(s * RPS, 8), RPS), :],
        out_hbm.at[pl.ds(pl.multiple_of(c * RN + s * RPS, 8), RPS), :],
    )


def _sc_degree(dm):
    @functools.partial(
        pl.kernel,
        out_type=jax.ShapeDtypeStruct((2 * RN, 16), jnp.float32),
        mesh=_mesh(),
        compiler_params=pltpu.CompilerParams(use_tc_tiling_on_sc=False),
        scratch_types=[
            pltpu.VMEM((DSTG, CH), jnp.int32),
            pltpu.VMEM((CH, 16), jnp.float32),
            pltpu.VMEM((ZB, 16), jnp.float32),
            pltpu.VMEM_SHARED((RN, 16), jnp.float32),
        ],
    )
    def body(dm_hbm, out_hbm, idx_v, ones_v, zb_v, acc_sh):
        c = lax.axis_index("c")
        s = lax.axis_index("s")
        one = jnp.ones((16,), jnp.float32)

        def fill_ones(i, carry):
            ones_v[i, :] = one
            return carry

        lax.fori_loop(0, CH, fill_ones, 0)
        _zero_init(zb_v, acc_sh, s)
        plsc.subcore_barrier()

        w = c * 16 + s

        def stage(b, carry):
            row0 = pl.multiple_of(w * DTPW + b * DSTG, 8)
            pltpu.sync_copy(dm_hbm.at[pl.ds(row0, DSTG), :], idx_v)
            for j in range(DSTG):
                pltpu.sync_copy(ones_v, acc_sh.at[idx_v.at[j]], add=True)
            return carry

        lax.fori_loop(0, DNB, stage, 0)
        plsc.subcore_barrier()
        _writeback(acc_sh, out_hbm, c, s)

    return body(dm)


def _sc_scatter(sm, dm, g2n):
    @functools.partial(
        pl.kernel,
        out_type=jax.ShapeDtypeStruct((2 * RN, 16), jnp.float32),
        mesh=_mesh(),
        compiler_params=pltpu.CompilerParams(use_tc_tiling_on_sc=False),
        scratch_types=[
            pltpu.VMEM((STG, CH), jnp.int32),
            pltpu.VMEM((STG, CH), jnp.int32),
            pltpu.VMEM((NBUF, CH, 16), jnp.float32),
            pltpu.VMEM((ZB, 16), jnp.float32),
            pltpu.VMEM_SHARED((RN, 16), jnp.float32),
            pltpu.SemaphoreType.DMA,
        ],
    )
    def body(sm_hbm, dm_hbm, g_hbm, out_hbm, sidx, didx, rows_v, zb_v, acc_sh,
             gsem):
        c = lax.axis_index("c")
        s = lax.axis_index("s")
        _zero_init(zb_v, acc_sh, s)
        plsc.subcore_barrier()

        def gfire(j):
            pltpu.async_copy(g_hbm.at[sidx.at[j]], rows_v.at[j % NBUF], gsem)

        def gwait(j):
            pltpu.make_async_copy(
                g_hbm.at[sidx.at[j]], rows_v.at[j % NBUF], gsem
            ).wait()

        def stage(b, carry):
            srow = pl.multiple_of(c * NROW + s * TPC + b * STG, 8)
            drow = pl.multiple_of(s * TPC + b * STG, 8)
            pltpu.sync_copy(sm_hbm.at[pl.ds(srow, STG), :], sidx)
            pltpu.sync_copy(dm_hbm.at[pl.ds(drow, STG), :], didx)
            for j in range(DEPTH):
                gfire(j)
            for j in range(STG):
                gwait(j)
                if j + DEPTH < STG:
                    gfire(j + DEPTH)
                pltpu.sync_copy(
                    rows_v.at[j % NBUF], acc_sh.at[didx.at[j]], add=True
                )
            return carry

        lax.fori_loop(0, NB, stage, 0)
        plsc.subcore_barrier()
        _writeback(acc_sh, out_hbm, c, s)

    return body(sm, dm, g2n)


def _tc_embed(x, We1, be1, We2, be2):
    def body(x_ref, w1_ref, b1_ref, w2_ref, b2_ref, o_ref):
        h = jnp.tanh(
            jnp.dot(x_ref[...], w1_ref[...], preferred_element_type=jnp.float32)
            + b1_ref[...]
        )
        o_ref[...] = jnp.tanh(
            jnp.dot(h, w2_ref[...], preferred_element_type=jnp.float32) + b2_ref[...]
        )

    return pl.pallas_call(
        body,
        grid=(GRID,),
        in_specs=[
            pl.BlockSpec((BLK, 6), lambda i: (i, 0)),
            pl.BlockSpec((6, 64), lambda i: (0, 0)),
            pl.BlockSpec((1, 64), lambda i: (0, 0)),
            pl.BlockSpec((64, 32), lambda i: (0, 0)),
            pl.BlockSpec((1, 32), lambda i: (0, 0)),
        ],
        out_specs=pl.BlockSpec((BLK, 32), lambda i: (i, 0)),
        out_shape=jax.ShapeDtypeStruct((N, 32), jnp.float32),
    )(x, We1, be1, We2, be2)


def _tc_prep(h0, degp, Wg1h):
    def body(h_ref, p_ref, w_ref, g_ref, dinv_ref):
        deg = p_ref[0, :, 0:1] + p_ref[1, :, 0:1] + 1.0
        dinv = lax.rsqrt(deg)
        t = jnp.dot(h_ref[...], w_ref[0], preferred_element_type=jnp.float32)
        g_ref[...] = (t * dinv)[None]
        dinv_ref[...] = dinv

    return pl.pallas_call(
        body,
        grid=(GRID, 2),
        in_specs=[
            pl.BlockSpec((BLK, 32), lambda i, c: (i, 0)),
            pl.BlockSpec((2, BLK, 16), lambda i, c: (0, i, 0)),
            pl.BlockSpec((1, 32, 16), lambda i, c: (c, 0, 0)),
        ],
        out_specs=[
            pl.BlockSpec((1, BLK, 16), lambda i, c: (c, i, 0)),
            pl.BlockSpec((BLK, 1), lambda i, c: (i, 0)),
        ],
        out_shape=[
            jax.ShapeDtypeStruct((2, N, 16), jnp.float32),
            jax.ShapeDtypeStruct((N, 1), jnp.float32),
        ],
    )(h0, degp, Wg1h)


def _tc_comb(acc, g, dinv, bg, Wh):
    def body(a_ref, g_ref, dinv_ref, bg_ref, w_ref, g2_ref):
        accv = jnp.concatenate([a_ref[0], a_ref[1]], axis=1)
        gv = jnp.concatenate([g_ref[0], g_ref[1]], axis=1)
        hv = jnp.maximum(dinv_ref[...] * (accv + gv) + bg_ref[...], 0.0)
        t = jnp.dot(hv, w_ref[0], preferred_element_type=jnp.float32)
        g2_ref[...] = (t * dinv_ref[...])[None]

    return pl.pallas_call(
        body,
        grid=(GRID, 2),
        in_specs=[
            pl.BlockSpec((2, BLK, 16), lambda i, c: (0, i, 0)),
            pl.BlockSpec((2, BLK, 16), lambda i, c: (0, i, 0)),
            pl.BlockSpec((BLK, 1), lambda i, c: (i, 0)),
            pl.BlockSpec((1, 32), lambda i, c: (0, 0)),
            pl.BlockSpec((1, 32, 16), lambda i, c: (c, 0, 0)),
        ],
        out_specs=pl.BlockSpec((1, BLK, 16), lambda i, c: (c, i, 0)),
        out_shape=jax.ShapeDtypeStruct((2, N, 16), jnp.float32),
    )(acc, g, dinv, bg, Wh)


def _tc_final(acc, g, dinv, bg2, Wp1, bp1, Wp2, bp2):
    def body(a_ref, g_ref, dinv_ref, bg_ref, w1_ref, b1_ref, w2_ref, b2_ref, o_ref):
        accv = jnp.concatenate([a_ref[0], a_ref[1]], axis=1)
        gv = jnp.concatenate([g_ref[0], g_ref[1]], axis=1)
        h2 = jnp.maximum(dinv_ref[...] * (accv + gv) + bg_ref[...], 0.0)
        a1 = jnp.tanh(
            jnp.dot(h2, w1_ref[...], preferred_element_type=jnp.float32) + b1_ref[...]
        )
        o_ref[...] = jnp.tanh(
            jnp.dot(a1, w2_ref[...], preferred_element_type=jnp.float32) + b2_ref[...]
        )

    return pl.pallas_call(
        body,
        grid=(GRID,),
        in_specs=[
            pl.BlockSpec((2, BLK, 16), lambda i: (0, i, 0)),
            pl.BlockSpec((2, BLK, 16), lambda i: (0, i, 0)),
            pl.BlockSpec((BLK, 1), lambda i: (i, 0)),
            pl.BlockSpec((1, 32), lambda i: (0, 0)),
            pl.BlockSpec((32, 32), lambda i: (0, 0)),
            pl.BlockSpec((1, 32), lambda i: (0, 0)),
            pl.BlockSpec((32, 1), lambda i: (0, 0)),
            pl.BlockSpec((1, 1), lambda i: (0, 0)),
        ],
        out_specs=pl.BlockSpec((BLK, 1), lambda i: (i, 0)),
        out_shape=jax.ShapeDtypeStruct((N, 1), jnp.float32),
    )(acc, g, dinv, bg2, Wp1, bp1, Wp2, bp2)


def kernel(x, edge_index, We1, be1, We2, be2, Wg1, bg1, Wg2, bg2, Wp1, bp1, Wp2, bp2):
    src = edge_index[0]
    dst = edge_index[1]
    srcp = jnp.concatenate([src, jnp.zeros((PAD,), jnp.int32)])
    dstp = jnp.concatenate([dst, jnp.full((PAD,), N, jnp.int32)])
    sm = jnp.concatenate([srcp, srcp + N]).reshape(2 * NROW, CH)
    dm = dstp.reshape(NROW, CH)

    Wg1h = Wg1.reshape(32, 2, 16).transpose(1, 0, 2)
    Wg2h = Wg2.reshape(32, 2, 16).transpose(1, 0, 2)

    degp = _sc_degree(dm).reshape(2, RN, 16)
    h0 = _tc_embed(x, We1, be1.reshape(1, 64), We2, be2.reshape(1, 32))
    g1, dinv = _tc_prep(h0, degp, Wg1h)

    acc1 = _sc_scatter(sm, dm, g1.reshape(2 * N, 16)).reshape(2, RN, 16)
    g2 = _tc_comb(acc1, g1, dinv, bg1.reshape(1, 32), Wg2h)
    acc2 = _sc_scatter(sm, dm, g2.reshape(2 * N, 16)).reshape(2, RN, 16)
    return _tc_final(
        acc2, g2, dinv, bg2.reshape(1, 32), Wp1, bp1.reshape(1, 32),
        Wp2, bp2.reshape(1, 1),
    )


# speedup vs baseline: 25.4741x; 1.0450x over previous
import functools

import jax
import jax.numpy as jnp
from jax import lax
from jax.experimental import pallas as pl
from jax.experimental.pallas import tpu as pltpu
from jax.experimental.pallas import tpu_sc as plsc

N = 100000
E = 1600000
CH = 128
TPC = 784
EP = 16 * TPC * CH
PAD = EP - E
NROW = EP // CH
STG = 16
NB = TPC // STG
NBUF = 8
DEPTH = 4
DTPW = NROW // 32
DSTG = 8
DNB = DTPW // DSTG
RN = 100096
RPS = RN // 16
ZB = 368

BLK = 4000
GRID = N // BLK


def _mesh():
    return plsc.VectorSubcoreMesh(
        core_axis_name="c", subcore_axis_name="s", num_cores=2, num_subcores=16
    )


def _zero_init(zb_v, acc_sh, s):
    zero = jnp.zeros((16,), jnp.float32)

    def fill_zero(i, carry):
        zb_v[i, :] = zero
        return carry

    lax.fori_loop(0, ZB, fill_zero, 0)

    def zcopy(k, carry):
        pltpu.sync_copy(
            zb_v, acc_sh.at[pl.ds(pl.multiple_of(s * RPS + k * ZB, 8), ZB), :]
        )
        return carry

    lax.fori_loop(0, 17, zcopy, 0)


def _writeback(acc_sh, out_hbm, c, s):
    pltpu.sync_copy(
        acc_sh.at[pl.ds(pl.multiple_of(s * RPS, 8), RPS), :],
        out_hbm.at[pl.ds(pl.multiple_of(c * RN + s * RPS, 8), RPS), :],
    )


def _sc_degree(dm):
    @functools.partial(
        pl.kernel,
        out_type=jax.ShapeDtypeStruct((2 * RN, 16), jnp.float32),
        mesh=_mesh(),
        compiler_params=pltpu.CompilerParams(use_tc_tiling_on_sc=False),
        scratch_types=[
            pltpu.VMEM((DSTG, CH), jnp.int32),
            pltpu.VMEM((CH, 16), jnp.float32),
            pltpu.VMEM((ZB, 16), jnp.float32),
            pltpu.VMEM_SHARED((RN, 16), jnp.float32),
        ],
    )
    def body(dm_hbm, out_hbm, idx_v, ones_v, zb_v, acc_sh):
        c = lax.axis_index("c")
        s = lax.axis_index("s")
        one = jnp.ones((16,), jnp.float32)

        def fill_ones(i, carry):
            ones_v[i, :] = one
            return carry

        lax.fori_loop(0, CH, fill_ones, 0)
        _zero_init(zb_v, acc_sh, s)
        plsc.subcore_barrier()

        w = c * 16 + s

        def stage(b, carry):
            row0 = pl.multiple_of(w * DTPW + b * DSTG, 8)
            pltpu.sync_copy(dm_hbm.at[pl.ds(row0, DSTG), :], idx_v)
            for j in range(DSTG):
                pltpu.sync_copy(ones_v, acc_sh.at[idx_v.at[j]], add=True)
            return carry

        lax.fori_loop(0, DNB, stage, 0)
        plsc.subcore_barrier()
        _writeback(acc_sh, out_hbm, c, s)

    return body(dm)


def _sc_scatter(sm, dm, g2n):
    @functools.partial(
        pl.kernel,
        out_type=jax.ShapeDtypeStruct((2 * RN, 16), jnp.float32),
        mesh=_mesh(),
        compiler_params=pltpu.CompilerParams(use_tc_tiling_on_sc=False),
        scratch_types=[
            pltpu.VMEM((STG, CH), jnp.int32),
            pltpu.VMEM((STG, CH), jnp.int32),
            pltpu.VMEM((NBUF, CH, 16), jnp.float32),
            pltpu.VMEM((ZB, 16), jnp.float32),
            pltpu.VMEM_SHARED((RN, 16), jnp.float32),
            pltpu.SemaphoreType.DMA,
        ],
    )
    def body(sm_hbm, dm_hbm, g_hbm, out_hbm, sidx, didx, rows_v, zb_v, acc_sh,
             gsem):
        c = lax.axis_index("c")
        s = lax.axis_index("s")
        _zero_init(zb_v, acc_sh, s)
        plsc.subcore_barrier()

        def gfire(j):
            pltpu.async_copy(g_hbm.at[sidx.at[j]], rows_v.at[j % NBUF], gsem)

        def gwait(j):
            pltpu.make_async_copy(
                g_hbm.at[sidx.at[j]], rows_v.at[j % NBUF], gsem
            ).wait()

        def stage(b, carry):
            srow = pl.multiple_of(c * NROW + s * TPC + b * STG, 8)
            drow = pl.multiple_of(s * TPC + b * STG, 8)
            pltpu.sync_copy(sm_hbm.at[pl.ds(srow, STG), :], sidx)
            pltpu.sync_copy(dm_hbm.at[pl.ds(drow, STG), :], didx)
            for j in range(DEPTH):
                gfire(j)
            for j in range(STG):
                gwait(j)
                if j + DEPTH < STG:
                    gfire(j + DEPTH)
                pltpu.sync_copy(
                    rows_v.at[j % NBUF], acc_sh.at[didx.at[j]], add=True
                )
            return carry

        lax.fori_loop(0, NB, stage, 0)
        plsc.subcore_barrier()
        _writeback(acc_sh, out_hbm, c, s)

    return body(sm, dm, g2n)


def _tc_embed(x, We1, be1, We2, be2):
    def body(x_ref, w1_ref, b1_ref, w2_ref, b2_ref, o_ref):
        h = jnp.tanh(
            jnp.dot(x_ref[...], w1_ref[...], preferred_element_type=jnp.float32)
            + b1_ref[...]
        )
        o_ref[...] = jnp.tanh(
            jnp.dot(h, w2_ref[...], preferred_element_type=jnp.float32) + b2_ref[...]
        )

    return pl.pallas_call(
        body,
        grid=(GRID,),
        in_specs=[
            pl.BlockSpec((BLK, 6), lambda i: (i, 0)),
            pl.BlockSpec((6, 64), lambda i: (0, 0)),
            pl.BlockSpec((1, 64), lambda i: (0, 0)),
            pl.BlockSpec((64, 32), lambda i: (0, 0)),
            pl.BlockSpec((1, 32), lambda i: (0, 0)),
        ],
        out_specs=pl.BlockSpec((BLK, 32), lambda i: (i, 0)),
        out_shape=jax.ShapeDtypeStruct((N, 32), jnp.float32),
    )(x, We1, be1, We2, be2)


def _tc_prep(h0, degp, Wg1h):
    def body(h_ref, p_ref, w_ref, g_ref, dinv_ref):
        deg = p_ref[0, :, 0:1] + p_ref[1, :, 0:1] + 1.0
        dinv = lax.rsqrt(deg)
        t = jnp.dot(h_ref[...], w_ref[0], preferred_element_type=jnp.float32)
        g_ref[...] = (t * dinv)[None]
        dinv_ref[...] = dinv

    return pl.pallas_call(
        body,
        grid=(GRID, 2),
        in_specs=[
            pl.BlockSpec((BLK, 32), lambda i, c: (i, 0)),
            pl.BlockSpec((2, BLK, 16), lambda i, c: (0, i, 0)),
            pl.BlockSpec((1, 32, 16), lambda i, c: (c, 0, 0)),
        ],
        out_specs=[
            pl.BlockSpec((1, BLK, 16), lambda i, c: (c, i, 0)),
            pl.BlockSpec((BLK, 1), lambda i, c: (i, 0)),
        ],
        out_shape=[
            jax.ShapeDtypeStruct((2, N, 16), jnp.float32),
            jax.ShapeDtypeStruct((N, 1), jnp.float32),
        ],
    )(h0, degp, Wg1h)


def _tc_comb(acc, g, dinv, bg, Wh):
    def body(a_ref, g_ref, dinv_ref, bg_ref, w_ref, g2_ref):
        accv = jnp.concatenate([a_ref[0], a_ref[1]], axis=1)
        gv = jnp.concatenate([g_ref[0], g_ref[1]], axis=1)
        hv = jnp.maximum(dinv_ref[...] * (accv + gv) + bg_ref[...], 0.0)
        t = jnp.dot(hv, w_ref[0], preferred_element_type=jnp.float32)
        g2_ref[...] = (t * dinv_ref[...])[None]

    return pl.pallas_call(
        body,
        grid=(GRID, 2),
        in_specs=[
            pl.BlockSpec((2, BLK, 16), lambda i, c: (0, i, 0)),
            pl.BlockSpec((2, BLK, 16), lambda i, c: (0, i, 0)),
            pl.BlockSpec((BLK, 1), lambda i, c: (i, 0)),
            pl.BlockSpec((1, 32), lambda i, c: (0, 0)),
            pl.BlockSpec((1, 32, 16), lambda i, c: (c, 0, 0)),
        ],
        out_specs=pl.BlockSpec((1, BLK, 16), lambda i, c: (c, i, 0)),
        out_shape=jax.ShapeDtypeStruct((2, N, 16), jnp.float32),
    )(acc, g, dinv, bg, Wh)


def _tc_final(acc, g, dinv, bg2, Wp1, bp1, Wp2, bp2):
    def body(a_ref, g_ref, dinv_ref, bg_ref, w1_ref, b1_ref, w2_ref, b2_ref, o_ref):
        accv = jnp.concatenate([a_ref[0], a_ref[1]], axis=1)
        gv = jnp.concatenate([g_ref[0], g_ref[1]], axis=1)
        h2 = jnp.maximum(dinv_ref[...] * (accv + gv) + bg_ref[...], 0.0)
        a1 = jnp.tanh(
            jnp.dot(h2, w1_ref[...], preferred_element_type=jnp.float32) + b1_ref[...]
        )
        o_ref[...] = jnp.tanh(
            jnp.dot(a1, w2_ref[...], preferred_element_type=jnp.float32) + b2_ref[...]
        )

    return pl.pallas_call(
        body,
        grid=(GRID,),
        in_specs=[
            pl.BlockSpec((2, BLK, 16), lambda i: (0, i, 0)),
            pl.BlockSpec((2, BLK, 16), lambda i: (0, i, 0)),
            pl.BlockSpec((BLK, 1), lambda i: (i, 0)),
            pl.BlockSpec((1, 32), lambda i: (0, 0)),
            pl.BlockSpec((32, 32), lambda i: (0, 0)),
            pl.BlockSpec((1, 32), lambda i: (0, 0)),
            pl.BlockSpec((32, 1), lambda i: (0, 0)),
            pl.BlockSpec((1, 1), lambda i: (0, 0)),
        ],
        out_specs=pl.BlockSpec((BLK, 1), lambda i: (i, 0)),
        out_shape=jax.ShapeDtypeStruct((N, 1), jnp.float32),
    )(acc, g, dinv, bg2, Wp1, bp1, Wp2, bp2)


def kernel(x, edge_index, We1, be1, We2, be2, Wg1, bg1, Wg2, bg2, Wp1, bp1, Wp2, bp2):
    src = edge_index[0]
    dst = edge_index[1]
    srcp = jnp.concatenate([src, jnp.zeros((PAD,), jnp.int32)])
    dstp = jnp.concatenate([dst, jnp.full((PAD,), N, jnp.int32)])
    sm = jnp.concatenate([srcp, srcp + N]).reshape(2 * NROW, CH)
    dm = dstp.reshape(NROW, CH)

    Wg1h = Wg1.reshape(32, 2, 16).transpose(1, 0, 2)
    Wg2h = Wg2.reshape(32, 2, 16).transpose(1, 0, 2)

    degp = _sc_degree(dm).reshape(2, RN, 16)
    h0 = _tc_embed(x, We1, be1.reshape(1, 64), We2, be2.reshape(1, 32))
    g1, dinv = _tc_prep(h0, degp, Wg1h)

    acc1 = _sc_scatter(sm, dm, g1.reshape(2 * N, 16)).reshape(2, RN, 16)
    g2 = _tc_comb(acc1, g1, dinv, bg1.reshape(1, 32), Wg2h)
    acc2 = _sc_scatter(sm, dm, g2.reshape(2 * N, 16)).reshape(2, RN, 16)
    return _tc_final(
        acc2, g2, dinv, bg2.reshape(1, 32), Wp1, bp1.reshape(1, 32),
        Wp2, bp2.reshape(1, 1),
    )
